# baseline (device time: 96698 ns/iter reference)
import jax
import jax.numpy as jnp
from jax import lax
from jax.experimental import pallas as pl
from jax.experimental.pallas import tpu as pltpu

N_DEV = 4


def kernel(x, router_W, route_idx, expert_W, shared_W):
    n_tokens, d_model = x.shape
    e_local, _, d_out = expert_W.shape
    n_experts = router_W.shape[-1]
    C = n_tokens // N_DEV
    H = C // 2

    xb = x.astype(jnp.bfloat16)
    rwb = router_W.astype(jnp.bfloat16)
    swb = shared_W.astype(jnp.bfloat16)

    T, B = 0, 10

    def body(x_ref, rw_ref, idx_ref, ew_ref, sw_ref, out_ref,
             comm_ref, probs_ref, ewb_ref, send_sems, recv_sems):
        s = pl.program_id(0)
        my_pos = lax.axis_index("i")
        left = lax.rem(my_pos + (N_DEV - 1), N_DEV)
        right = lax.rem(my_pos + 1, N_DEV)

        q = jnp.where(s < e_local, 0, s - (e_local - 1))
        a_q = lax.rem(my_pos + (N_DEV - q), N_DEV)
        b_q = lax.rem(my_pos + q, N_DEV)
        rows_T = pl.ds(a_q * C, H)
        rows_B = pl.ds(b_q * C + H, H)

        col = lax.broadcasted_iota(jnp.int32, (H, n_experts), 1)

        def masked_x(rows, e):
            pe = jnp.sum(jnp.where(col == e, probs_ref[rows, :], 0.0),
                         axis=-1, keepdims=True)
            coef = jnp.where(idx_ref[rows, :] == e, pe, 0.0)
            return x_ref[rows, :] * coef.astype(jnp.bfloat16)

        @pl.when(s == 0)
        def _entry():
            barrier_sem = pltpu.get_barrier_semaphore()
            for nbr in (left, right):
                pl.semaphore_signal(
                    barrier_sem, inc=1,
                    device_id=(nbr,), device_id_type=pl.DeviceIdType.MESH,
                )
            pl.semaphore_wait(barrier_sem, 2)
            scores = jnp.dot(x_ref[...], rw_ref[...],
                             preferred_element_type=jnp.float32)
            m = jnp.max(scores, axis=-1, keepdims=True)
            p = jnp.exp(scores - m)
            probs_ref[...] = p / jnp.sum(p, axis=-1, keepdims=True)

        @pl.when(s < e_local)
        def _pass0_step():
            ewb_ref[pl.ds(s * d_model, d_model), :] = (
                ew_ref[0].astype(jnp.bfloat16))
            e = my_pos * e_local + s
            w_j = ewb_ref[pl.ds(s * d_model, d_model), :]
            c_T = jnp.dot(masked_x(rows_T, e), w_j,
                          preferred_element_type=jnp.float32)
            c_B = jnp.dot(masked_x(rows_B, e), w_j,
                          preferred_element_type=jnp.float32)

            @pl.when(s == 0)
            def _init():
                out_ref[rows_T, :] = c_T.astype(jnp.bfloat16)
                out_ref[rows_B, :] = c_B.astype(jnp.bfloat16)

            @pl.when(s > 0)
            def _acc():
                out_ref[rows_T, :] = (out_ref[rows_T, :]
                                      + c_T).astype(jnp.bfloat16)
                out_ref[rows_B, :] = (out_ref[rows_B, :]
                                      + c_B).astype(jnp.bfloat16)

        @pl.when(s >= e_local)
        def _fused_pass():
            def fused(rows):
                xm = jnp.concatenate(
                    [masked_x(rows, my_pos * e_local + jj)
                     for jj in range(e_local)], axis=1)
                return jnp.dot(xm, ewb_ref[...],
                               preferred_element_type=jnp.float32)

            out_ref[rows_T, :] = fused(rows_T).astype(jnp.bfloat16)
            out_ref[rows_B, :] = fused(rows_B).astype(jnp.bfloat16)

        def mk(src_slot, dst_slot, sem, dev):
            return pltpu.make_async_remote_copy(
                src_ref=comm_ref.at[src_slot],
                dst_ref=comm_ref.at[dst_slot],
                send_sem=send_sems.at[sem],
                recv_sem=recv_sems.at[sem],
                device_id=(dev,),
                device_id_type=pl.DeviceIdType.MESH,
            )

        @pl.when(s == e_local - 1)
        def _rs0():
            comm_ref[T + 0, :, :] = out_ref[rows_T, :]
            mk(T + 0, T + 3, 0, right).start()
            comm_ref[B + 0, :, :] = out_ref[rows_B, :]
            mk(B + 0, B + 3, 6, left).start()

        @pl.when(s == e_local)
        def _rs1():
            mk(T + 0, T + 3, 0, right).wait()
            comm_ref[T + 1, :, :] = (comm_ref[T + 3, :, :].astype(jnp.float32)
                                     + out_ref[rows_T, :]).astype(jnp.bfloat16)
            mk(T + 1, T + 4, 1, right).start()
            mk(B + 0, B + 3, 6, left).wait()
            comm_ref[B + 1, :, :] = (comm_ref[B + 3, :, :].astype(jnp.float32)
                                     + out_ref[rows_B, :]).astype(jnp.bfloat16)
            mk(B + 1, B + 4, 7, left).start()

        @pl.when(s == e_local + 1)
        def _rs2():
            mk(T + 1, T + 4, 1, right).wait()
            comm_ref[T + 2, :, :] = (comm_ref[T + 4, :, :].astype(jnp.float32)
                                     + out_ref[rows_T, :]).astype(jnp.bfloat16)
            mk(T + 2, T + 5, 2, right).start()
            mk(B + 1, B + 4, 7, left).wait()
            comm_ref[B + 2, :, :] = (comm_ref[B + 4, :, :].astype(jnp.float32)
                                     + out_ref[rows_B, :]).astype(jnp.bfloat16)
            mk(B + 2, B + 5, 8, left).start()

        @pl.when(s == e_local + 2)
        def _rs3_and_ag():
            sw = sw_ref[...]

            mk(T + 2, T + 5, 2, right).wait()
            comm_ref[T + 6, :, :] = (
                comm_ref[T + 5, :, :].astype(jnp.float32)
                + out_ref[rows_T, :]
                + jnp.dot(x_ref[rows_T, :], sw,
                          preferred_element_type=jnp.float32)
            ).astype(jnp.bfloat16)
            mk(B + 2, B + 5, 8, left).wait()
            comm_ref[B + 6, :, :] = (
                comm_ref[B + 5, :, :].astype(jnp.float32)
                + out_ref[rows_B, :]
                + jnp.dot(x_ref[rows_B, :], sw,
                          preferred_element_type=jnp.float32)
            ).astype(jnp.bfloat16)

            def t_rows(c):
                return pl.ds(c * C, H)

            def b_rows(c):
                return pl.ds(c * C + H, H)

            cp1 = lax.rem(my_pos + 1, N_DEV)
            cp2 = lax.rem(my_pos + 2, N_DEV)
            cm1 = lax.rem(my_pos + (N_DEV - 1), N_DEV)

            rt0 = mk(T + 6, T + 7, 3, right)
            lb0 = mk(B + 6, B + 7, 9, left)
            rt0.start()
            lb0.start()
            out_ref[t_rows(cp1), :] = comm_ref[T + 6, :, :]
            out_ref[b_rows(cm1), :] = comm_ref[B + 6, :, :]
            rt0.wait()
            lb0.wait()

            rt1 = mk(T + 7, T + 8, 4, right)
            lb1 = mk(B + 7, B + 8, 10, left)
            rt1.start()
            lb1.start()
            out_ref[t_rows(my_pos), :] = comm_ref[T + 7, :, :]
            out_ref[b_rows(my_pos), :] = comm_ref[B + 7, :, :]
            rt1.wait()
            lb1.wait()

            rt2 = mk(T + 8, T + 9, 5, right)
            lb2 = mk(B + 8, B + 9, 11, left)
            rt2.start()
            lb2.start()
            out_ref[t_rows(cm1), :] = comm_ref[T + 8, :, :]
            out_ref[b_rows(cp1), :] = comm_ref[B + 8, :, :]
            rt2.wait()
            lb2.wait()

            out_ref[t_rows(cp2), :] = comm_ref[T + 9, :, :]
            out_ref[b_rows(cp2), :] = comm_ref[B + 9, :, :]

    return pl.pallas_call(
        body,
        grid=(e_local + N_DEV - 1,),
        out_shape=jax.ShapeDtypeStruct((n_tokens, d_out), jnp.bfloat16),
        in_specs=[
            pl.BlockSpec((n_tokens, d_model), lambda s: (0, 0)),
            pl.BlockSpec((d_model, n_experts), lambda s: (0, 0)),
            pl.BlockSpec((n_tokens, 1), lambda s: (0, 0)),
            pl.BlockSpec((1, d_model, d_out),
                         lambda s: (jnp.where(s < 8, s, 7), 0, 0)),
            pl.BlockSpec((d_model, d_out), lambda s: (0, 0)),
        ],
        out_specs=pl.BlockSpec((n_tokens, d_out), lambda s: (0, 0)),
        scratch_shapes=[
            pltpu.VMEM((20, H, d_out), jnp.bfloat16),
            pltpu.VMEM((n_tokens, n_experts), jnp.float32),
            pltpu.VMEM((e_local * d_model, d_out), jnp.bfloat16),
            pltpu.SemaphoreType.DMA((12,)),
            pltpu.SemaphoreType.DMA((12,)),
        ],
        compiler_params=pltpu.CompilerParams(
            collective_id=0,
            dimension_semantics=("arbitrary",),
            vmem_limit_bytes=63 * 1024 * 1024,
        ),
    )(xb, rwb, route_idx, expert_W, swb)


# device time: 93081 ns/iter; 1.0389x vs baseline; 1.0389x over previous
import jax
import jax.numpy as jnp
from jax import lax
from jax.experimental import pallas as pl
from jax.experimental.pallas import tpu as pltpu

N_DEV = 4


def kernel(x, router_W, route_idx, expert_W, shared_W):
    n_tokens, d_model = x.shape
    e_local, _, d_out = expert_W.shape
    n_experts = router_W.shape[-1]
    C = n_tokens // N_DEV
    H = C // 2

    rwb = router_W.astype(jnp.bfloat16)
    swb = shared_W.astype(jnp.bfloat16)

    T, B = 0, 10

    def body(x_ref, rw_ref, idx_ref, ew_ref, sw_ref, out_ref,
             comm_ref, probs_ref, ewb_ref, send_sems, recv_sems):
        s = pl.program_id(0)
        my_pos = lax.axis_index("i")
        left = lax.rem(my_pos + (N_DEV - 1), N_DEV)
        right = lax.rem(my_pos + 1, N_DEV)

        q = jnp.where(s < e_local, 0, s - (e_local - 1))
        a_q = lax.rem(my_pos + (N_DEV - q), N_DEV)
        b_q = lax.rem(my_pos + q, N_DEV)
        rows_T = pl.ds(a_q * C, H)
        rows_B = pl.ds(b_q * C + H, H)

        col = lax.broadcasted_iota(jnp.int32, (H, n_experts), 1)

        def masked_x(rows, e):
            pe = jnp.sum(jnp.where(col == e, probs_ref[rows, :], 0.0),
                         axis=-1, keepdims=True)
            coef = jnp.where(idx_ref[rows, :] == e, pe, 0.0)
            return (x_ref[rows, :] * coef).astype(jnp.bfloat16)

        @pl.when(s == 0)
        def _entry():
            barrier_sem = pltpu.get_barrier_semaphore()
            for nbr in (left, right):
                pl.semaphore_signal(
                    barrier_sem, inc=1,
                    device_id=(nbr,), device_id_type=pl.DeviceIdType.MESH,
                )
            pl.semaphore_wait(barrier_sem, 2)
            scores = jnp.dot(x_ref[...].astype(jnp.bfloat16), rw_ref[...],
                             preferred_element_type=jnp.float32)
            m = jnp.max(scores, axis=-1, keepdims=True)
            p = jnp.exp(scores - m)
            probs_ref[...] = p / jnp.sum(p, axis=-1, keepdims=True)

        @pl.when(s < e_local)
        def _pass0_step():
            ewb_ref[pl.ds(s * d_model, d_model), :] = (
                ew_ref[0].astype(jnp.bfloat16))
            e = my_pos * e_local + s
            w_j = ewb_ref[pl.ds(s * d_model, d_model), :]
            c_T = jnp.dot(masked_x(rows_T, e), w_j,
                          preferred_element_type=jnp.float32)
            c_B = jnp.dot(masked_x(rows_B, e), w_j,
                          preferred_element_type=jnp.float32)

            @pl.when(s == 0)
            def _init():
                out_ref[rows_T, :] = c_T.astype(jnp.bfloat16)
                out_ref[rows_B, :] = c_B.astype(jnp.bfloat16)

            @pl.when(s > 0)
            def _acc():
                out_ref[rows_T, :] = (out_ref[rows_T, :]
                                      + c_T).astype(jnp.bfloat16)
                out_ref[rows_B, :] = (out_ref[rows_B, :]
                                      + c_B).astype(jnp.bfloat16)

        @pl.when(s >= e_local)
        def _fused_pass():
            def fused(rows):
                xm = jnp.concatenate(
                    [masked_x(rows, my_pos * e_local + jj)
                     for jj in range(e_local)], axis=1)
                return jnp.dot(xm, ewb_ref[...],
                               preferred_element_type=jnp.float32)

            out_ref[rows_T, :] = fused(rows_T).astype(jnp.bfloat16)
            out_ref[rows_B, :] = fused(rows_B).astype(jnp.bfloat16)

        def mk(src_slot, dst_slot, sem, dev):
            return pltpu.make_async_remote_copy(
                src_ref=comm_ref.at[src_slot],
                dst_ref=comm_ref.at[dst_slot],
                send_sem=send_sems.at[sem],
                recv_sem=recv_sems.at[sem],
                device_id=(dev,),
                device_id_type=pl.DeviceIdType.MESH,
            )

        @pl.when(s == e_local - 1)
        def _rs0():
            comm_ref[T + 0, :, :] = out_ref[rows_T, :]
            mk(T + 0, T + 3, 0, right).start()
            comm_ref[B + 0, :, :] = out_ref[rows_B, :]
            mk(B + 0, B + 3, 6, left).start()

        @pl.when(s == e_local)
        def _rs1():
            mk(T + 0, T + 3, 0, right).wait()
            comm_ref[T + 1, :, :] = (comm_ref[T + 3, :, :].astype(jnp.float32)
                                     + out_ref[rows_T, :]).astype(jnp.bfloat16)
            mk(T + 1, T + 4, 1, right).start()
            mk(B + 0, B + 3, 6, left).wait()
            comm_ref[B + 1, :, :] = (comm_ref[B + 3, :, :].astype(jnp.float32)
                                     + out_ref[rows_B, :]).astype(jnp.bfloat16)
            mk(B + 1, B + 4, 7, left).start()

        @pl.when(s == e_local + 1)
        def _rs2():
            mk(T + 1, T + 4, 1, right).wait()
            comm_ref[T + 2, :, :] = (comm_ref[T + 4, :, :].astype(jnp.float32)
                                     + out_ref[rows_T, :]).astype(jnp.bfloat16)
            mk(T + 2, T + 5, 2, right).start()
            mk(B + 1, B + 4, 7, left).wait()
            comm_ref[B + 2, :, :] = (comm_ref[B + 4, :, :].astype(jnp.float32)
                                     + out_ref[rows_B, :]).astype(jnp.bfloat16)
            mk(B + 2, B + 5, 8, left).start()

        @pl.when(s == e_local + 2)
        def _rs3_and_ag():
            sw = sw_ref[...]

            mk(T + 2, T + 5, 2, right).wait()
            comm_ref[T + 6, :, :] = (
                comm_ref[T + 5, :, :].astype(jnp.float32)
                + out_ref[rows_T, :]
                + jnp.dot(x_ref[rows_T, :].astype(jnp.bfloat16), sw,
                          preferred_element_type=jnp.float32)
            ).astype(jnp.bfloat16)
            mk(B + 2, B + 5, 8, left).wait()
            comm_ref[B + 6, :, :] = (
                comm_ref[B + 5, :, :].astype(jnp.float32)
                + out_ref[rows_B, :]
                + jnp.dot(x_ref[rows_B, :].astype(jnp.bfloat16), sw,
                          preferred_element_type=jnp.float32)
            ).astype(jnp.bfloat16)

            def t_rows(c):
                return pl.ds(c * C, H)

            def b_rows(c):
                return pl.ds(c * C + H, H)

            cp1 = lax.rem(my_pos + 1, N_DEV)
            cp2 = lax.rem(my_pos + 2, N_DEV)
            cm1 = lax.rem(my_pos + (N_DEV - 1), N_DEV)

            rt0 = mk(T + 6, T + 7, 3, right)
            lb0 = mk(B + 6, B + 7, 9, left)
            rt0.start()
            lb0.start()
            out_ref[t_rows(cp1), :] = comm_ref[T + 6, :, :]
            out_ref[b_rows(cm1), :] = comm_ref[B + 6, :, :]
            rt0.wait()
            lb0.wait()

            rt1 = mk(T + 7, T + 8, 4, right)
            lb1 = mk(B + 7, B + 8, 10, left)
            rt1.start()
            lb1.start()
            out_ref[t_rows(my_pos), :] = comm_ref[T + 7, :, :]
            out_ref[b_rows(my_pos), :] = comm_ref[B + 7, :, :]
            rt1.wait()
            lb1.wait()

            rt2 = mk(T + 8, T + 9, 5, right)
            lb2 = mk(B + 8, B + 9, 11, left)
            rt2.start()
            lb2.start()
            out_ref[t_rows(cm1), :] = comm_ref[T + 8, :, :]
            out_ref[b_rows(cp1), :] = comm_ref[B + 8, :, :]
            rt2.wait()
            lb2.wait()

            out_ref[t_rows(cp2), :] = comm_ref[T + 9, :, :]
            out_ref[b_rows(cp2), :] = comm_ref[B + 9, :, :]

    return pl.pallas_call(
        body,
        grid=(e_local + N_DEV - 1,),
        out_shape=jax.ShapeDtypeStruct((n_tokens, d_out), jnp.bfloat16),
        in_specs=[
            pl.BlockSpec((n_tokens, d_model), lambda s: (0, 0)),
            pl.BlockSpec((d_model, n_experts), lambda s: (0, 0)),
            pl.BlockSpec((n_tokens, 1), lambda s: (0, 0)),
            pl.BlockSpec((1, d_model, d_out),
                         lambda s: (jnp.where(s < 8, s, 7), 0, 0)),
            pl.BlockSpec((d_model, d_out), lambda s: (0, 0)),
        ],
        out_specs=pl.BlockSpec((n_tokens, d_out), lambda s: (0, 0)),
        scratch_shapes=[
            pltpu.VMEM((20, H, d_out), jnp.bfloat16),
            pltpu.VMEM((n_tokens, n_experts), jnp.float32),
            pltpu.VMEM((e_local * d_model, d_out), jnp.bfloat16),
            pltpu.SemaphoreType.DMA((12,)),
            pltpu.SemaphoreType.DMA((12,)),
        ],
        compiler_params=pltpu.CompilerParams(
            collective_id=0,
            dimension_semantics=("arbitrary",),
            vmem_limit_bytes=63 * 1024 * 1024,
        ),
    )(x, rwb, route_idx, expert_W, swb)
